# baseline (device time: 2974201 ns/iter reference)
import jax
import jax.numpy as jnp
from jax import lax
from jax.experimental import pallas as pl
from jax.experimental.pallas import tpu as pltpu

CH = 128


def kernel(x, dest):
    m, n = x.shape
    max_chunks = m // CH

    my_z = lax.axis_index("z")

    d0 = (dest == 0).astype(jnp.int32)
    a_cum = jnp.cumsum(d0)
    b_cum = jnp.cumsum(1 - d0)
    c0 = a_cum[-1].astype(jnp.int32)
    s = jnp.arange(m, dtype=jnp.int32)
    perm = jnp.where(
        s < c0,
        jnp.searchsorted(a_cum, s + 1, side="left"),
        jnp.searchsorted(b_cum, s - c0 + 1, side="left"),
    ).astype(jnp.int32)

    k_rows = jnp.where(my_z == 0, m - c0, c0).astype(jnp.int32)
    send_off = jnp.where(my_z == 0, c0, 0).astype(jnp.int32)
    send_buf = x[jnp.roll(perm, -send_off)].astype(jnp.bfloat16)
    scalars = jnp.stack([c0, k_rows, send_off]).astype(jnp.int32)

    def body(sc_ref, send_ref, out_ref, recv_ref, send_sems, recv_sems):
        zi = lax.axis_index("z")
        xi = lax.axis_index("x")
        yi = lax.axis_index("y")
        peer = (xi, yi, 1 - zi)
        c0s = sc_ref[0]
        ks = sc_ref[1]
        soff = sc_ref[2]
        n_chunks = (ks + CH - 1) // CH

        barrier_sem = pltpu.get_barrier_semaphore()
        pl.semaphore_signal(
            barrier_sem, inc=1, device_id=peer,
            device_id_type=pl.DeviceIdType.MESH,
        )
        pl.semaphore_wait(barrier_sem, 1)

        def chunk_rdma(i):
            o = pl.multiple_of(i * CH, CH)
            return pltpu.make_async_remote_copy(
                src_ref=send_ref.at[pl.ds(o, CH)],
                dst_ref=recv_ref.at[pl.ds(o, CH)],
                send_sem=send_sems.at[i],
                recv_sem=recv_sems.at[i],
                device_id=peer,
                device_id_type=pl.DeviceIdType.MESH,
            )

        def send_body(i, _):
            chunk_rdma(i).start()
            return 0

        lax.fori_loop(0, n_chunks, send_body, 0)

        def recv_body(i, _):
            chunk_rdma(i).wait_recv()
            return 0

        lax.fori_loop(0, n_chunks, recv_body, 0)

        xs_v = pltpu.roll(send_ref[:, :], soff, 0)
        rolled = pltpu.roll(recv_ref[:, :], jnp.where(zi == 0, c0s, 0), 0)
        rowid = lax.broadcasted_iota(jnp.int32, (m, n), 0)
        take_xs = (rowid < c0s) == (zi == 0)
        out_ref[:, :] = jnp.where(take_xs, xs_v, rolled)

        def wait_send_body(i, _):
            chunk_rdma(i).wait_send()
            return 0

        lax.fori_loop(0, n_chunks, wait_send_body, 0)

    return pl.pallas_call(
        body,
        out_shape=jax.ShapeDtypeStruct((m, n), jnp.bfloat16),
        in_specs=[
            pl.BlockSpec(memory_space=pltpu.SMEM),
            pl.BlockSpec(memory_space=pltpu.VMEM),
        ],
        out_specs=pl.BlockSpec(memory_space=pltpu.VMEM),
        scratch_shapes=[
            pltpu.VMEM((m, n), jnp.bfloat16),
            pltpu.SemaphoreType.DMA((max_chunks,)),
            pltpu.SemaphoreType.DMA((max_chunks,)),
        ],
        compiler_params=pltpu.CompilerParams(collective_id=0),
    )(scalars, send_buf)


# device time: 53661 ns/iter; 55.4257x vs baseline; 55.4257x over previous
import jax
import jax.numpy as jnp
from jax import lax
from jax.experimental import pallas as pl
from jax.experimental.pallas import tpu as pltpu

CH = 128


def kernel(x, dest):
    m, n = x.shape
    max_chunks = m // CH

    my_z = lax.axis_index("z")

    perm = jnp.argsort(dest, stable=True).astype(jnp.int32)
    c0 = jnp.sum(dest == 0).astype(jnp.int32)

    k_rows = jnp.where(my_z == 0, m - c0, c0).astype(jnp.int32)
    send_off = jnp.where(my_z == 0, c0, 0).astype(jnp.int32)
    send_buf = x[jnp.roll(perm, -send_off)].astype(jnp.bfloat16)
    scalars = jnp.stack([c0, k_rows, send_off]).astype(jnp.int32)

    def body(sc_ref, send_ref, out_ref, recv_ref, send_sems, recv_sems):
        zi = lax.axis_index("z")
        xi = lax.axis_index("x")
        yi = lax.axis_index("y")
        peer = (xi, yi, 1 - zi)
        c0s = sc_ref[0]
        ks = sc_ref[1]
        soff = sc_ref[2]
        n_chunks = (ks + CH - 1) // CH

        barrier_sem = pltpu.get_barrier_semaphore()
        pl.semaphore_signal(
            barrier_sem, inc=1, device_id=peer,
            device_id_type=pl.DeviceIdType.MESH,
        )
        pl.semaphore_wait(barrier_sem, 1)

        def chunk_rdma(i):
            o = pl.multiple_of(i * CH, CH)
            return pltpu.make_async_remote_copy(
                src_ref=send_ref.at[pl.ds(o, CH)],
                dst_ref=recv_ref.at[pl.ds(o, CH)],
                send_sem=send_sems.at[i],
                recv_sem=recv_sems.at[i],
                device_id=peer,
                device_id_type=pl.DeviceIdType.MESH,
            )

        def send_body(i, _):
            chunk_rdma(i).start()
            return 0

        lax.fori_loop(0, n_chunks, send_body, 0)

        def recv_body(i, _):
            chunk_rdma(i).wait_recv()
            return 0

        lax.fori_loop(0, n_chunks, recv_body, 0)

        xs_v = pltpu.roll(send_ref[:, :], soff, 0)
        rolled = pltpu.roll(recv_ref[:, :], jnp.where(zi == 0, c0s, 0), 0)
        rowid = lax.broadcasted_iota(jnp.int32, (m, n), 0)
        take_xs = (rowid < c0s) == (zi == 0)
        out_ref[:, :] = jnp.where(take_xs, xs_v, rolled)

        def wait_send_body(i, _):
            chunk_rdma(i).wait_send()
            return 0

        lax.fori_loop(0, n_chunks, wait_send_body, 0)

    return pl.pallas_call(
        body,
        out_shape=jax.ShapeDtypeStruct((m, n), jnp.bfloat16),
        in_specs=[
            pl.BlockSpec(memory_space=pltpu.SMEM),
            pl.BlockSpec(memory_space=pltpu.VMEM),
        ],
        out_specs=pl.BlockSpec(memory_space=pltpu.VMEM),
        scratch_shapes=[
            pltpu.VMEM((m, n), jnp.bfloat16),
            pltpu.SemaphoreType.DMA((max_chunks,)),
            pltpu.SemaphoreType.DMA((max_chunks,)),
        ],
        compiler_params=pltpu.CompilerParams(collective_id=0),
    )(scalars, send_buf)


# device time: 47593 ns/iter; 62.4924x vs baseline; 1.1275x over previous
import jax
import jax.numpy as jnp
from jax import lax
from jax.experimental import pallas as pl
from jax.experimental.pallas import tpu as pltpu

CH = 128


def kernel(x, dest):
    m, n = x.shape
    mp = m + CH
    max_chunks = mp // CH

    perm = jnp.argsort(dest, stable=True).astype(jnp.int32)
    perm_pad = jnp.concatenate([perm, jnp.zeros((CH,), jnp.int32)])
    xs_pad = x[perm_pad].astype(jnp.bfloat16)
    c0 = jnp.sum(dest == 0).astype(jnp.int32).reshape(1)

    def body(c0_ref, xs_ref, out_ref, recv_ref, send_sems, recv_sems):
        zi = lax.axis_index("z")
        xi = lax.axis_index("x")
        yi = lax.axis_index("y")
        peer = (xi, yi, 1 - zi)
        c0s = c0_ref[0]

        ks = jnp.where(zi == 0, m - c0s, c0s)
        soff = jnp.where(zi == 0, c0s, 0)
        a = pl.multiple_of((soff // 8) * 8, 8)
        r = soff - a
        r_peer = jnp.where(zi == 0, 0, (m - c0s) % 8)
        n_send_chunks = (ks + r + CH - 1) // CH
        n_recv_chunks = (ks + r_peer + CH - 1) // CH

        barrier_sem = pltpu.get_barrier_semaphore()
        pl.semaphore_signal(
            barrier_sem, inc=1, device_id=peer,
            device_id_type=pl.DeviceIdType.MESH,
        )
        pl.semaphore_wait(barrier_sem, 1)

        def chunk_rdma(i):
            o = pl.multiple_of(i * CH, CH)
            return pltpu.make_async_remote_copy(
                src_ref=xs_ref.at[pl.ds(pl.multiple_of(a + o, 8), CH)],
                dst_ref=recv_ref.at[pl.ds(o, CH)],
                send_sem=send_sems.at[i],
                recv_sem=recv_sems.at[i],
                device_id=peer,
                device_id_type=pl.DeviceIdType.MESH,
            )

        def send_body(i, _):
            chunk_rdma(i).start()
            return 0

        lax.fori_loop(0, n_send_chunks, send_body, 0)

        def recv_body(i, _):
            chunk_rdma(i).wait_recv()
            return 0

        lax.fori_loop(0, n_recv_chunks, recv_body, 0)

        shift = jnp.where(zi == 0, c0s - r_peer, mp - r_peer)
        rolled = pltpu.roll(recv_ref[:, :], shift, 0)
        rowid = lax.broadcasted_iota(jnp.int32, (m, n), 0)
        take_xs = (rowid < c0s) == (zi == 0)
        out_ref[:, :] = jnp.where(
            take_xs, xs_ref[pl.ds(0, m), :], rolled[:m, :])

        def wait_send_body(i, _):
            chunk_rdma(i).wait_send()
            return 0

        lax.fori_loop(0, n_send_chunks, wait_send_body, 0)

    return pl.pallas_call(
        body,
        out_shape=jax.ShapeDtypeStruct((m, n), jnp.bfloat16),
        in_specs=[
            pl.BlockSpec(memory_space=pltpu.SMEM),
            pl.BlockSpec(memory_space=pltpu.VMEM),
        ],
        out_specs=pl.BlockSpec(memory_space=pltpu.VMEM),
        scratch_shapes=[
            pltpu.VMEM((mp, n), jnp.bfloat16),
            pltpu.SemaphoreType.DMA((max_chunks,)),
            pltpu.SemaphoreType.DMA((max_chunks,)),
        ],
        compiler_params=pltpu.CompilerParams(collective_id=0),
    )(c0, xs_pad)
